# baseline (speedup 1.0000x reference)
import jax
import jax.numpy as jnp
from jax import lax
from jax.experimental import pallas as pl
from jax.experimental.pallas import tpu as pltpu
from jax.experimental.pallas import tpu_sc as plsc

NC = 2
NS = 16
NW = NC * NS
LANES = 16
DEPTH = 4

B = 4096
T = 200
D = 64
N = B * T
PER_W = N // NW
CH = 128
N_CHUNKS = PER_W // CH
CPT = B // CH
VPR = D // LANES


def _sc_body(idx_hbm, pos_hbm, alpha_hbm, sym_hbm, con_hbm, out_hbm,
             idx_all, alpha_v, *rest):
    syms = rest[0:DEPTH]
    cons = rest[DEPTH:2 * DEPTH]
    outs = rest[2 * DEPTH:3 * DEPTH]
    poss = rest[3 * DEPTH:4 * DEPTH]
    gss = rest[4 * DEPTH:5 * DEPTH]
    oss = rest[5 * DEPTH:6 * DEPTH]

    wid = lax.axis_index("s") * NC + lax.axis_index("c")
    gc0 = wid * N_CHUNKS

    pltpu.sync_copy(idx_hbm.at[pl.ds(gc0, N_CHUNKS)], idx_all)
    pltpu.sync_copy(alpha_hbm, alpha_v)
    a_v = alpha_v[...]

    def issue_gathers(c, b):
        t_c = (gc0 + c) // CPT
        pltpu.async_copy(sym_hbm.at[idx_all.at[c]], syms[b], gss[b])
        pltpu.async_copy(con_hbm.at[idx_all.at[c]], cons[b], gss[b])
        pltpu.async_copy(pos_hbm.at[pl.ds(t_c * D, D)], poss[b], gss[b])

    def wait_gathers(c, b):
        pltpu.make_async_copy(sym_hbm.at[idx_all.at[c]], syms[b], gss[b]).wait()
        pltpu.make_async_copy(con_hbm.at[idx_all.at[c]], cons[b], gss[b]).wait()
        t_c = (gc0 + c) // CPT
        pltpu.make_async_copy(pos_hbm.at[pl.ds(t_c * D, D)], poss[b], gss[b]).wait()

    def out_slice(c):
        gc = gc0 + c
        t_c = gc // CPT
        b0 = (gc % CPT) * CH
        return out_hbm.at[pl.ds(b0, CH), t_c]

    for k in range(DEPTH):
        issue_gathers(k, k)

    @pl.loop(0, N_CHUNKS, step=DEPTH)
    def outer(c_base):
        for b in range(DEPTH):
            c = c_base + b

            wait_gathers(c, b)

            @pl.when(c >= DEPTH)
            def _():
                pltpu.make_async_copy(outs[b], out_slice(c), oss[b]).wait()

            p = [poss[b][pl.ds(j * LANES, LANES)] for j in range(VPR)]
            sym_b, con_b, out_b = syms[b], cons[b], outs[b]

            @pl.loop(0, CH)
            def row_body(i):
                for j in range(VPR):
                    sl = pl.ds(j * LANES, LANES)
                    out_b[i, sl] = sym_b[i, sl] + a_v * con_b[i, sl] + p[j]

            pltpu.async_copy(outs[b], out_slice(c), oss[b])

            @pl.when(c + DEPTH < N_CHUNKS)
            def _():
                issue_gathers(c + DEPTH, b)

    for k in range(DEPTH):
        c = N_CHUNKS - DEPTH + k
        pltpu.make_async_copy(outs[k], out_slice(c), oss[k]).wait()


@jax.jit
def _run(idx_t, pos_flat, alpha_vec, symbol_emb, concept_emb):
    mesh = plsc.VectorSubcoreMesh(
        core_axis_name="c", subcore_axis_name="s",
        num_cores=NC, num_subcores=NS)
    buf = lambda: pltpu.VMEM((CH, D), jnp.float32)
    posb = lambda: pltpu.VMEM((D,), jnp.float32)
    return pl.kernel(
        _sc_body,
        out_type=jax.ShapeDtypeStruct((B, T, D), jnp.float32),
        mesh=mesh,
        compiler_params=pltpu.CompilerParams(use_tc_tiling_on_sc=False),
        scratch_types=(
            [pltpu.VMEM((N_CHUNKS, CH), jnp.int32),
             pltpu.VMEM((LANES,), jnp.float32)]
            + [buf() for _ in range(3 * DEPTH)]
            + [posb() for _ in range(DEPTH)]
            + [pltpu.SemaphoreType.DMA for _ in range(2 * DEPTH)]
        ),
    )(idx_t, pos_flat, alpha_vec, symbol_emb, concept_emb)


def kernel(idx, symbol_emb, concept_emb, pos_emb, alpha):
    idx_t = idx.T.astype(jnp.int32).reshape(N // CH, CH)
    pos_flat = pos_emb.reshape(T * D)
    alpha_vec = jnp.full((LANES,), alpha, dtype=jnp.float32)
    return _run(idx_t, pos_flat, alpha_vec, symbol_emb, concept_emb)
